# Initial kernel scaffold; baseline (speedup 1.0000x reference)
import jax
import jax.numpy as jnp
from jax.experimental import pallas as pl


def kernel(hidden_states, scale, W):
    raise NotImplementedError("write your pallas kernel here")


# speedup vs baseline: 1.8106x; 1.8106x over previous
import functools

import jax
import jax.numpy as jnp
from jax.experimental import pallas as pl
from jax.experimental.pallas import tpu as pltpu

B, S, D, E = 4, 2048, 2048, 64
EPS = 1e-06


def _router_body(x_ref, scale_ref, w_ref, routes_ref, p_ref):
    x = x_ref[...]
    ssq = jnp.sum(x * x, axis=1, keepdims=True)
    r = jax.lax.rsqrt(ssq * (1.0 / D) + EPS)
    xn = (x * r) * scale_ref[...]
    logits = jax.lax.dot_general(
        xn, w_ref[...],
        dimension_numbers=(((1,), (1,)), ((), ())),
        preferred_element_type=jnp.float32,
    )
    m = jnp.max(logits, axis=1, keepdims=True)
    ids = jax.lax.broadcasted_iota(jnp.int32, logits.shape, 1)
    idx = jnp.min(jnp.where(logits == m, ids, E), axis=1)
    denom = jnp.sum(jnp.exp(logits - m), axis=1)
    routes_ref[0, 0, :] = idx
    p_ref[0, 0, :] = 1.0 / denom


@functools.partial(jax.jit, static_argnames=())
def kernel(hidden_states, scale, W):
    T = hidden_states.shape[0] * hidden_states.shape[1]
    d = hidden_states.shape[2]
    x = hidden_states.reshape(T, d)
    TB = 512
    G = T // TB
    routes2, p2 = pl.pallas_call(
        _router_body,
        grid=(G,),
        in_specs=[
            pl.BlockSpec((TB, d), lambda i: (i, 0)),
            pl.BlockSpec((1, d), lambda i: (0, 0)),
            pl.BlockSpec((E, d), lambda i: (0, 0)),
        ],
        out_specs=[
            pl.BlockSpec((1, 1, TB), lambda i: (i, 0, 0)),
            pl.BlockSpec((1, 1, TB), lambda i: (i, 0, 0)),
        ],
        out_shape=[
            jax.ShapeDtypeStruct((G, 1, TB), jnp.int32),
            jax.ShapeDtypeStruct((G, 1, TB), jnp.float32),
        ],
        compiler_params=pltpu.CompilerParams(
            dimension_semantics=("parallel",),
        ),
    )(x, scale.reshape(1, d), W)
    return routes2.reshape(T), p2.reshape(T)


# speedup vs baseline: 2.3859x; 1.3177x over previous
import functools

import jax
import jax.numpy as jnp
from jax.experimental import pallas as pl
from jax.experimental.pallas import tpu as pltpu

B, S, D, E = 4, 2048, 2048, 64
EPS = 1e-06


def _router_body(x_ref, scale_ref, w_ref, routes_ref, p_ref):
    x = x_ref[...]
    ssq = jnp.sum(x * x, axis=1, keepdims=True)
    r = jax.lax.rsqrt(ssq * (1.0 / D) + EPS)
    xn = x * r
    ws = w_ref[...] * scale_ref[...]
    logits = jax.lax.dot_general(
        ws, xn,
        dimension_numbers=(((1,), (1,)), ((), ())),
        preferred_element_type=jnp.float32,
    )
    m = jnp.max(logits, axis=0, keepdims=True)
    ids = jax.lax.broadcasted_iota(jnp.int32, logits.shape, 0)
    idx = jnp.min(jnp.where(logits == m, ids, E), axis=0)
    denom = jnp.sum(jnp.exp(logits - m), axis=0)
    routes_ref[0, 0, :] = idx
    p_ref[0, 0, :] = 1.0 / denom


@functools.partial(jax.jit, static_argnames=())
def kernel(hidden_states, scale, W):
    T = hidden_states.shape[0] * hidden_states.shape[1]
    d = hidden_states.shape[2]
    x = hidden_states.reshape(T, d)
    TB = 512
    G = T // TB
    routes2, p2 = pl.pallas_call(
        _router_body,
        grid=(G,),
        in_specs=[
            pl.BlockSpec((TB, d), lambda i: (i, 0)),
            pl.BlockSpec((1, d), lambda i: (0, 0)),
            pl.BlockSpec((E, d), lambda i: (0, 0)),
        ],
        out_specs=[
            pl.BlockSpec((1, 1, TB), lambda i: (i, 0, 0)),
            pl.BlockSpec((1, 1, TB), lambda i: (i, 0, 0)),
        ],
        out_shape=[
            jax.ShapeDtypeStruct((G, 1, TB), jnp.int32),
            jax.ShapeDtypeStruct((G, 1, TB), jnp.float32),
        ],
        compiler_params=pltpu.CompilerParams(
            dimension_semantics=("parallel",),
        ),
    )(x, scale.reshape(1, d), W)
    return routes2.reshape(T), p2.reshape(T)


# speedup vs baseline: 2.6952x; 1.1297x over previous
import functools

import jax
import jax.numpy as jnp
from jax.experimental import pallas as pl
from jax.experimental.pallas import tpu as pltpu

B, S, D, E = 4, 2048, 2048, 64
EPS = 1e-06


def _router_body(x_ref, scale_ref, w_ref, routes_ref, p_ref):
    x = x_ref[...]
    ssq = jnp.sum(x * x, axis=1, keepdims=True)
    r = jax.lax.rsqrt(ssq * (1.0 / D) + EPS)
    xn = x * r
    ws = w_ref[...] * scale_ref[...]
    logits = jax.lax.dot_general(
        ws, xn,
        dimension_numbers=(((1,), (1,)), ((), ())),
        preferred_element_type=jnp.float32,
    )
    m = jnp.max(logits, axis=0, keepdims=True)
    ids = jax.lax.broadcasted_iota(jnp.int32, logits.shape, 0)
    idx = jnp.min(jnp.where(logits == m, ids, E), axis=0)
    denom = jnp.sum(jnp.exp(logits - m), axis=0)
    routes_ref[0, 0, :] = idx
    p_ref[0, 0, :] = 1.0 / denom


@functools.partial(jax.jit, static_argnames=())
def kernel(hidden_states, scale, W):
    T = hidden_states.shape[0] * hidden_states.shape[1]
    d = hidden_states.shape[2]
    x = hidden_states.reshape(T, d)
    TB = 1024
    G = T // TB
    routes2, p2 = pl.pallas_call(
        _router_body,
        grid=(G,),
        in_specs=[
            pl.BlockSpec((TB, d), lambda i: (i, 0)),
            pl.BlockSpec((1, d), lambda i: (0, 0)),
            pl.BlockSpec((E, d), lambda i: (0, 0)),
        ],
        out_specs=[
            pl.BlockSpec((1, 1, TB), lambda i: (i, 0, 0)),
            pl.BlockSpec((1, 1, TB), lambda i: (i, 0, 0)),
        ],
        out_shape=[
            jax.ShapeDtypeStruct((G, 1, TB), jnp.int32),
            jax.ShapeDtypeStruct((G, 1, TB), jnp.float32),
        ],
        compiler_params=pltpu.CompilerParams(
            dimension_semantics=("parallel",),
        ),
    )(x, scale.reshape(1, d), W)
    return routes2.reshape(T), p2.reshape(T)


# speedup vs baseline: 2.7549x; 1.0221x over previous
import functools

import jax
import jax.numpy as jnp
from jax.experimental import pallas as pl
from jax.experimental.pallas import tpu as pltpu

B, S, D, E = 4, 2048, 2048, 64
EPS = 1e-06


def _router_body(x_ref, scale_ref, w_ref, routes_ref, p_ref):
    x = x_ref[...]
    ssq = jnp.sum(x * x, axis=1, keepdims=True)
    r = jax.lax.rsqrt(ssq * (1.0 / D) + EPS)
    xn = x * r
    ws = w_ref[...] * scale_ref[...]
    logits = jax.lax.dot_general(
        ws, xn,
        dimension_numbers=(((1,), (1,)), ((), ())),
        preferred_element_type=jnp.float32,
    )
    m = jnp.max(logits, axis=0, keepdims=True)
    ids = jax.lax.broadcasted_iota(jnp.int32, logits.shape, 0)
    idx = jnp.min(jnp.where(logits == m, ids, E), axis=0)
    denom = jnp.sum(jnp.exp(logits - m), axis=0)
    routes_ref[0, 0, :] = idx
    p_ref[0, 0, :] = 1.0 / denom


@functools.partial(jax.jit, static_argnames=())
def kernel(hidden_states, scale, W):
    T = hidden_states.shape[0] * hidden_states.shape[1]
    d = hidden_states.shape[2]
    x = hidden_states.reshape(T, d)
    TB = 2048
    G = T // TB
    routes2, p2 = pl.pallas_call(
        _router_body,
        grid=(G,),
        in_specs=[
            pl.BlockSpec((TB, d), lambda i: (i, 0)),
            pl.BlockSpec((1, d), lambda i: (0, 0)),
            pl.BlockSpec((E, d), lambda i: (0, 0)),
        ],
        out_specs=[
            pl.BlockSpec((1, 1, TB), lambda i: (i, 0, 0)),
            pl.BlockSpec((1, 1, TB), lambda i: (i, 0, 0)),
        ],
        out_shape=[
            jax.ShapeDtypeStruct((G, 1, TB), jnp.int32),
            jax.ShapeDtypeStruct((G, 1, TB), jnp.float32),
        ],
        compiler_params=pltpu.CompilerParams(
            dimension_semantics=("parallel",),
        ),
    )(x, scale.reshape(1, d), W)
    return routes2.reshape(T), p2.reshape(T)
